# Initial kernel scaffold; baseline (speedup 1.0000x reference)
import jax
import jax.numpy as jnp
from jax.experimental import pallas as pl


def kernel(node_feat, edge_index, phi1, W1, b1, phi_hidden, phi2, W2, b2):
    raise NotImplementedError("write your pallas kernel here")


# speedup vs baseline: 19.0802x; 19.0802x over previous
import functools

import jax
import jax.numpy as jnp
from jax import lax
from jax.experimental import pallas as pl
from jax.experimental.pallas import tpu as pltpu
from jax.experimental.pallas import tpu_sc as plsc

N = 10000
D = 128
E = 320000
NC = 2
NS = 16
NT = NC * NS
EPT = E // NT
CHUNK = 125
NCHUNK = EPT // CHUNK
ROWS_PER_TILE = N // NS
PADN = 640 * NS

_mesh = plsc.VectorSubcoreMesh(core_axis_name="c", subcore_axis_name="s")


def _deg_body(col3, row3, eq3, ones_h, zer_h,
              degp, selfp, degq, selfq,
              colbuf, rowbuf, eqbuf, onesb, zb, deg_s, self_s):
    cc = lax.axis_index("c")
    ss = lax.axis_index("s")
    tile = cc * NS + ss
    pltpu.sync_copy(zer_h, zb)
    pltpu.sync_copy(ones_h, onesb)
    pltpu.sync_copy(zb, deg_s.at[pl.ds(ss * 640, 640)])
    pltpu.sync_copy(zb, self_s.at[pl.ds(ss * 640, 640)])
    pltpu.sync_copy(col3.at[tile], colbuf)
    pltpu.sync_copy(row3.at[tile], rowbuf)
    pltpu.sync_copy(eq3.at[tile], eqbuf)
    plsc.subcore_barrier()

    def body(j, _):
        pltpu.sync_copy(onesb, deg_s.at[colbuf.at[j]], add=True)
        pltpu.sync_copy(eqbuf.at[j], self_s.at[rowbuf.at[j]], add=True)
        return _

    lax.fori_loop(0, NCHUNK, body, None)
    plsc.subcore_barrier()

    @pl.when(cc == 0)
    def _():
        pltpu.sync_copy(deg_s.at[pl.ds(ss * 640, 640)],
                        degp.at[pl.ds(ss * 640, 640)])
        pltpu.sync_copy(self_s.at[pl.ds(ss * 640, 640)],
                        selfp.at[pl.ds(ss * 640, 640)])

    @pl.when(cc == 1)
    def _():
        pltpu.sync_copy(deg_s.at[pl.ds(ss * 640, 640)],
                        degq.at[pl.ds(ss * 640, 640)])
        pltpu.sync_copy(self_s.at[pl.ds(ss * 640, 640)],
                        selfq.at[pl.ds(ss * 640, 640)])


_deg_kernel = pl.kernel(
    _deg_body,
    out_type=(jax.ShapeDtypeStruct((PADN,), jnp.float32),
              jax.ShapeDtypeStruct((PADN,), jnp.float32),
              jax.ShapeDtypeStruct((PADN,), jnp.float32),
              jax.ShapeDtypeStruct((PADN,), jnp.float32)),
    mesh=_mesh,
    scratch_types=[
        pltpu.VMEM((NCHUNK, CHUNK), jnp.int32),
        pltpu.VMEM((NCHUNK, CHUNK), jnp.int32),
        pltpu.VMEM((NCHUNK, CHUNK), jnp.float32),
        pltpu.VMEM((CHUNK,), jnp.float32),
        pltpu.VMEM((640,), jnp.float32),
        pltpu.VMEM_SHARED((PADN,), jnp.float32),
        pltpu.VMEM_SHARED((PADN,), jnp.float32),
    ],
)


def _spmm_body(xp, col3, row3, zrows,
               tp,
               colbuf, rowbuf, gbuf, ys):
    cc = lax.axis_index("c")
    ss = lax.axis_index("s")
    tile = cc * NS + ss
    pltpu.sync_copy(zrows, gbuf)
    for i in range(ROWS_PER_TILE // CHUNK):
        pltpu.sync_copy(gbuf, ys.at[pl.ds(ss * ROWS_PER_TILE + i * CHUNK, CHUNK)])
    pltpu.sync_copy(col3.at[tile], colbuf)
    pltpu.sync_copy(row3.at[tile], rowbuf)
    plsc.subcore_barrier()

    def body(j, _):
        pltpu.sync_copy(xp.at[colbuf.at[j]], gbuf)
        pltpu.sync_copy(gbuf, ys.at[rowbuf.at[j]], add=True)
        return _

    lax.fori_loop(0, NCHUNK, body, None)
    plsc.subcore_barrier()
    pltpu.sync_copy(ys.at[pl.ds(ss * ROWS_PER_TILE, ROWS_PER_TILE)],
                    tp.at[cc, ss])


_spmm_kernel = pl.kernel(
    _spmm_body,
    out_type=jax.ShapeDtypeStruct((NC, NS, ROWS_PER_TILE, D), jnp.float32),
    mesh=_mesh,
    scratch_types=[
        pltpu.VMEM((NCHUNK, CHUNK), jnp.int32),
        pltpu.VMEM((NCHUNK, CHUNK), jnp.int32),
        pltpu.VMEM((CHUNK, D), jnp.float32),
        pltpu.VMEM_SHARED((N, D), jnp.float32),
    ],
)


BR = 1000


def _dense_body(x_ref, t_ref, c_ref, d_ref, phi_ref, w_ref, b_ref,
                out_ref, outp_ref, *, relu):
    t = t_ref[0] + t_ref[1]
    phi = phi_ref[...]
    u = x_ref[...] * (1.0 - c_ref[...] * phi) + t * (d_ref[...] * phi)
    h = jnp.dot(u, w_ref[...], preferred_element_type=jnp.float32) + b_ref[...]
    if relu:
        h = jnp.maximum(h, 0.0)
    out_ref[...] = h
    if outp_ref is not None:
        outp_ref[...] = h * d_ref[...]


def _mid_body(x_ref, t_ref, c_ref, d_ref, phi_ref, out_ref, outp_ref):
    t = t_ref[0] + t_ref[1]
    phi = phi_ref[...]
    h = x_ref[...] * (1.0 - c_ref[...] * phi) + t * (d_ref[...] * phi)
    out_ref[...] = h
    outp_ref[...] = h * d_ref[...]


_bs_x = pl.BlockSpec((BR, D), lambda i: (i, 0))
_bs_t = pl.BlockSpec((NC, BR, D), lambda i: (0, i, 0))
_bs_n1 = pl.BlockSpec((BR, 1), lambda i: (i, 0))
_bs_row = pl.BlockSpec((1, D), lambda i: (0, 0))
_bs_w = pl.BlockSpec((D, D), lambda i: (0, 0))

_dense1 = pl.pallas_call(
    functools.partial(_dense_body, relu=True),
    grid=(N // BR,),
    in_specs=[_bs_x, _bs_t, _bs_n1, _bs_n1, _bs_row, _bs_w, _bs_row],
    out_specs=(_bs_x, _bs_x),
    out_shape=(jax.ShapeDtypeStruct((N, D), jnp.float32),
               jax.ShapeDtypeStruct((N, D), jnp.float32)),
)


def _dense_final_body(x_ref, t_ref, c_ref, d_ref, phi_ref, w_ref, b_ref,
                      out_ref):
    _dense_body(x_ref, t_ref, c_ref, d_ref, phi_ref, w_ref, b_ref,
                out_ref, None, relu=False)


_dense2 = pl.pallas_call(
    _dense_final_body,
    grid=(N // BR,),
    in_specs=[_bs_x, _bs_t, _bs_n1, _bs_n1, _bs_row, _bs_w, _bs_row],
    out_specs=_bs_x,
    out_shape=jax.ShapeDtypeStruct((N, D), jnp.float32),
)

_mid = pl.pallas_call(
    _mid_body,
    grid=(N // BR,),
    in_specs=[_bs_x, _bs_t, _bs_n1, _bs_n1, _bs_row],
    out_specs=(_bs_x, _bs_x),
    out_shape=(jax.ShapeDtypeStruct((N, D), jnp.float32),
               jax.ShapeDtypeStruct((N, D), jnp.float32)),
)


def kernel(node_feat, edge_index, phi1, W1, b1, phi_hidden, phi2, W2, b2):
    row3 = edge_index[0].reshape(NT, NCHUNK, CHUNK)
    col3 = edge_index[1].reshape(NT, NCHUNK, CHUNK)
    eq3 = (row3 == col3).astype(jnp.float32)
    ones_h = jnp.ones((CHUNK,), jnp.float32)
    zer_h = jnp.zeros((640,), jnp.float32)
    zrows = jnp.zeros((CHUNK, D), jnp.float32)

    degp, selfp, degq, selfq = _deg_kernel(col3, row3, eq3, ones_h, zer_h)
    deg = degp[:N] + degq[:N] + 1.0
    selfcnt = selfp[:N] + selfq[:N]
    dvec = lax.rsqrt(deg)
    cvec = (deg - 1.0) / deg + selfcnt
    c2 = cvec[:, None]
    d2 = dvec[:, None]

    x0 = node_feat
    x0p = x0 * d2

    def spmm_t(xp):
        return _spmm_kernel(xp, col3, row3, zrows).reshape(NC, N, D)

    t0 = spmm_t(x0p)
    x1, x1p = _dense1(x0, t0, c2, d2, phi1[None, :], W1, b1[None, :])
    t1 = spmm_t(x1p)
    x2, x2p = _mid(x1, t1, c2, d2, phi_hidden[0][None, :])
    t2 = spmm_t(x2p)
    x3, x3p = _mid(x2, t2, c2, d2, phi_hidden[1][None, :])
    t3 = spmm_t(x3p)
    out = _dense2(x3, t3, c2, d2, phi2[None, :], W2, b2[None, :])
    return out
